# baseline (speedup 1.0000x reference)
import jax
import jax.numpy as jnp
from jax.experimental import pallas as pl
from jax.experimental.pallas import tpu as pltpu

_DELTA_VAR = 0.5
_DELTA_DIST = 1.5
_ALPHA = 1.0
_BETA = 1.0
_GAMMA = 0.001
_NC = 16


def _batch_contrib(x, t, rj, tk, mg):
    lbl = jax.lax.broadcasted_iota(jnp.int32, (_NC, 1), 0)
    mask = (t == lbl).astype(jnp.float32)

    counts = jnp.sum(mask, axis=1, keepdims=True)
    safe_counts = jnp.maximum(counts, 1.0)
    present = counts > 0.0

    sums = jax.lax.dot_general(
        mask, x, dimension_numbers=(((1,), (1,)), ((), ())),
        preferred_element_type=jnp.float32)
    means = sums / safe_counts

    c_means = jax.lax.dot_general(
        means, mask, dimension_numbers=(((0,), (0,)), ((), ())),
        preferred_element_type=jnp.float32)

    dev = jnp.sum(jnp.abs(x - c_means), axis=0, keepdims=True)
    var = jnp.maximum(dev - _DELTA_VAR, 0.0) ** 2
    var_sums = jax.lax.dot_general(
        mask, var, dimension_numbers=(((1,), (1,)), ((), ())),
        preferred_element_type=jnp.float32)
    c_var = jnp.where(present, var_sums / safe_counts, 0.0)
    var_term = jnp.sum(c_var)

    mc = c_means[:, :_NC]
    mc_j = jax.lax.dot_general(
        mc, rj, dimension_numbers=(((1,), (0,)), ((), ())),
        preferred_element_type=jnp.float32)
    mc_k = jax.lax.dot_general(
        mc, tk, dimension_numbers=(((1,), (0,)), ((), ())),
        preferred_element_type=jnp.float32)
    d = jnp.sum(jnp.abs(mc_j - mc_k), axis=0, keepdims=True)
    presf = present.astype(jnp.float32)
    pres_j = jax.lax.dot_general(
        presf, rj, dimension_numbers=(((0,), (0,)), ((), ())),
        preferred_element_type=jnp.float32)
    pres_k = jax.lax.dot_general(
        presf, tk, dimension_numbers=(((0,), (0,)), ((), ())),
        preferred_element_type=jnp.float32)
    hinge = jnp.maximum(mg - d, 0.0) ** 2
    c_dist = jnp.sum(pres_j * pres_k * hinge)
    K = jnp.sum(presf)
    denom = jnp.maximum(K * (K - 1.0), 1.0)
    dist_term = jnp.where(K > 1.0, c_dist / denom, 0.0)

    col_norms = jnp.where(present, jnp.sum(jnp.abs(means), axis=1,
                                           keepdims=True), 0.0)
    reg_term = jnp.sum(col_norms) / K

    return (_ALPHA * var_term + _BETA * dist_term + _GAMMA * reg_term)


_NSPLIT = 4


def _loss_kernel(x_hbm, t_hbm, out_ref, xb, tb, semx, semt):
    b = pl.program_id(0)
    nb = pl.num_programs(0)
    F = xb.shape[1]
    Fq = F // _NSPLIT

    nn = _NC * _NC
    rows = jax.lax.broadcasted_iota(jnp.int32, (_NC, nn), 0)
    cols = jax.lax.broadcasted_iota(jnp.int32, (_NC, nn), 1)
    rj = (cols // _NC == rows).astype(jnp.float32)
    tk = (cols % _NC == rows).astype(jnp.float32)
    c1 = jax.lax.broadcasted_iota(jnp.int32, (1, nn), 1)
    mg = jnp.where(c1 // _NC == c1 % _NC, 0.0, 2.0 * _DELTA_DIST)

    def _copies(bb, slot):
        cps = []
        for s in range(_NSPLIT):
            cps.append(pltpu.make_async_copy(
                x_hbm.at[bb, pl.ds(s * Fq, Fq)],
                xb.at[slot, pl.ds(s * Fq, Fq)],
                semx.at[slot, s]))
        cps.append(pltpu.make_async_copy(
            t_hbm.at[bb], tb.at[slot], semt.at[slot]))
        return cps

    def _start(bb, slot):
        for cp in _copies(bb, slot):
            cp.start()

    def _wait(bb, slot):
        for cp in _copies(bb, slot):
            cp.wait()

    @pl.when(b == 0)
    def _prologue():
        out_ref[...] = jnp.zeros((1, 1), jnp.float32)
        _start(0, 0)

    def _step(slot):
        @pl.when(b + 1 < nb)
        def _prefetch():
            _start(b + 1, 1 - slot)

        _wait(b, slot)
        contrib = _batch_contrib(xb[slot], tb[slot], rj, tk, mg) / nb
        out_ref[...] += jnp.full((1, 1), contrib, jnp.float32)

    even = b % 2 == 0

    @pl.when(even)
    def _even():
        _step(0)

    @pl.when(jnp.logical_not(even))
    def _odd():
        _step(1)


def kernel(input, target):
    B, F, P = input.shape
    t3 = target.reshape(B, 1, P)
    out = pl.pallas_call(
        _loss_kernel,
        grid=(B,),
        in_specs=[
            pl.BlockSpec(memory_space=pl.ANY),
            pl.BlockSpec(memory_space=pl.ANY),
        ],
        out_specs=pl.BlockSpec((1, 1), lambda i: (0, 0)),
        out_shape=jax.ShapeDtypeStruct((1, 1), jnp.float32),
        scratch_shapes=[
            pltpu.VMEM((2, F, P), jnp.float32),
            pltpu.VMEM((2, 1, P), jnp.int32),
            pltpu.SemaphoreType.DMA((2, _NSPLIT)),
            pltpu.SemaphoreType.DMA((2,)),
        ],
    )(input, t3)
    return out[0, 0]


# speedup vs baseline: 1.0449x; 1.0449x over previous
import jax
import jax.numpy as jnp
from jax.experimental import pallas as pl

_DELTA_VAR = 0.5
_DELTA_DIST = 1.5
_ALPHA = 1.0
_BETA = 1.0
_GAMMA = 0.001
_NC = 16


def _batch_contrib(x, t, rj, tk, mg):
    lbl = jax.lax.broadcasted_iota(jnp.int32, (_NC, 1), 0)
    mask = (t == lbl).astype(jnp.float32)

    counts = jnp.sum(mask, axis=1, keepdims=True)
    safe_counts = jnp.maximum(counts, 1.0)
    present = counts > 0.0

    sums = jax.lax.dot_general(
        mask, x, dimension_numbers=(((1,), (1,)), ((), ())),
        preferred_element_type=jnp.float32)
    means = sums / safe_counts

    c_means = jax.lax.dot_general(
        means, mask, dimension_numbers=(((0,), (0,)), ((), ())),
        preferred_element_type=jnp.float32)

    dev = jnp.sum(jnp.abs(x - c_means), axis=0, keepdims=True)
    var = jnp.maximum(dev - _DELTA_VAR, 0.0) ** 2
    var_sums = jax.lax.dot_general(
        mask, var, dimension_numbers=(((1,), (1,)), ((), ())),
        preferred_element_type=jnp.float32)
    c_var = jnp.where(present, var_sums / safe_counts, 0.0)
    var_term = jnp.sum(c_var)

    mc = c_means[:, :_NC]
    mc_j = jax.lax.dot_general(
        mc, rj, dimension_numbers=(((1,), (0,)), ((), ())),
        preferred_element_type=jnp.float32)
    mc_k = jax.lax.dot_general(
        mc, tk, dimension_numbers=(((1,), (0,)), ((), ())),
        preferred_element_type=jnp.float32)
    d = jnp.sum(jnp.abs(mc_j - mc_k), axis=0, keepdims=True)
    presf = present.astype(jnp.float32)
    pres_j = jax.lax.dot_general(
        presf, rj, dimension_numbers=(((0,), (0,)), ((), ())),
        preferred_element_type=jnp.float32)
    pres_k = jax.lax.dot_general(
        presf, tk, dimension_numbers=(((0,), (0,)), ((), ())),
        preferred_element_type=jnp.float32)
    hinge = jnp.maximum(mg - d, 0.0) ** 2
    c_dist = jnp.sum(pres_j * pres_k * hinge)
    K = jnp.sum(presf)
    denom = jnp.maximum(K * (K - 1.0), 1.0)
    dist_term = jnp.where(K > 1.0, c_dist / denom, 0.0)

    col_norms = jnp.where(present, jnp.sum(jnp.abs(means), axis=1,
                                           keepdims=True), 0.0)
    reg_term = jnp.sum(col_norms) / K

    return (_ALPHA * var_term + _BETA * dist_term + _GAMMA * reg_term)


def _loss_kernel(x_ref, t_ref, out_ref):
    b = pl.program_id(0)
    nb = pl.num_programs(0)
    bpb = x_ref.shape[0]

    nn = _NC * _NC
    rows = jax.lax.broadcasted_iota(jnp.int32, (_NC, nn), 0)
    cols = jax.lax.broadcasted_iota(jnp.int32, (_NC, nn), 1)
    rj = (cols // _NC == rows).astype(jnp.float32)
    tk = (cols % _NC == rows).astype(jnp.float32)
    c1 = jax.lax.broadcasted_iota(jnp.int32, (1, nn), 1)
    mg = jnp.where(c1 // _NC == c1 % _NC, 0.0, 2.0 * _DELTA_DIST)

    contrib = 0.0
    for bb in range(bpb):
        contrib = contrib + _batch_contrib(
            x_ref[bb], t_ref[bb], rj, tk, mg)
    contrib = contrib / (nb * bpb)

    @pl.when(b == 0)
    def _():
        out_ref[...] = jnp.zeros((1, 1), jnp.float32)

    out_ref[...] += jnp.full((1, 1), contrib, jnp.float32)


def kernel(input, target):
    B, F, P = input.shape
    t3 = target.reshape(B, 1, P)
    bpb = 2 if B % 2 == 0 else 1
    out = pl.pallas_call(
        _loss_kernel,
        grid=(B // bpb,),
        in_specs=[
            pl.BlockSpec((bpb, F, P), lambda i: (i, 0, 0)),
            pl.BlockSpec((bpb, 1, P), lambda i: (i, 0, 0)),
        ],
        out_specs=pl.BlockSpec((1, 1), lambda i: (0, 0)),
        out_shape=jax.ShapeDtypeStruct((1, 1), jnp.float32),
    )(input, t3)
    return out[0, 0]
